# baseline (speedup 1.0000x reference)
import functools

import jax
import jax.numpy as jnp
from jax import lax
from jax.experimental import pallas as pl
from jax.experimental.pallas import tpu as pltpu
from jax.experimental.pallas import tpu_sc as plsc

N = 10000
IN_CH = 128
HID = 64
E = 320000

NC, NS = 2, 16
NW = NC * NS
BATCH = 125
EDGE_ROWS = E // BATCH
ROWS_PER_WORKER = EDGE_ROWS // NW
RCHUNK = 2
CHUNKS = ROWS_PER_WORKER // RCHUNK
NPAD = 10240
SLICE = NPAD // NS

_sc_mesh = plsc.VectorSubcoreMesh(core_axis_name="c", subcore_axis_name="s")


@functools.partial(
    pl.kernel,
    out_type=jax.ShapeDtypeStruct((NC, NPAD), jnp.float32),
    mesh=_sc_mesh,
    scratch_types=[
        pltpu.VMEM((ROWS_PER_WORKER, BATCH), jnp.int32),
        pltpu.VMEM((BATCH,), jnp.float32),
        pltpu.VMEM_SHARED((NPAD,), jnp.float32),
        pltpu.SemaphoreType.DMA,
    ],
    compiler_params=pltpu.CompilerParams(use_tc_tiling_on_sc=False),
)
def _deg_kernel(dst_hbm, zeros_hbm, ones_hbm, out_hbm, idx_v, ones_v, deg_sp,
                sem):
    c = lax.axis_index("c")
    s = lax.axis_index("s")
    row0 = (c * NS + s) * ROWS_PER_WORKER
    pltpu.sync_copy(dst_hbm.at[pl.ds(row0, ROWS_PER_WORKER)], idx_v)
    pltpu.sync_copy(zeros_hbm.at[pl.ds(0, SLICE)],
                    deg_sp.at[pl.ds(s * SLICE, SLICE)])
    pltpu.sync_copy(ones_hbm, ones_v)
    plsc.subcore_barrier()

    def fire(k, carry):
        pltpu.async_copy(ones_v, deg_sp.at[idx_v.at[k]], sem, add=True)
        return carry

    lax.fori_loop(0, ROWS_PER_WORKER, fire, 0)

    def drain(k, carry):
        pltpu.make_async_copy(ones_v, deg_sp.at[idx_v.at[k]], sem).wait()
        return carry

    lax.fori_loop(0, ROWS_PER_WORKER, drain, 0)
    plsc.subcore_barrier()
    pltpu.sync_copy(deg_sp.at[pl.ds(s * SLICE, SLICE)],
                    out_hbm.at[c].at[pl.ds(s * SLICE, SLICE)])


@functools.partial(
    pl.kernel,
    out_type=jax.ShapeDtypeStruct((NC, NPAD, HID), jnp.float32),
    mesh=_sc_mesh,
    scratch_types=[
        pltpu.VMEM((ROWS_PER_WORKER, BATCH), jnp.int32),
        pltpu.VMEM((ROWS_PER_WORKER, BATCH), jnp.int32),
        pltpu.VMEM((4, RCHUNK * BATCH, HID), jnp.float32),
        pltpu.VMEM_SHARED((NPAD, HID), jnp.float32),
        pltpu.SemaphoreType.DMA,
        pltpu.SemaphoreType.DMA,
        pltpu.SemaphoreType.DMA,
        pltpu.SemaphoreType.DMA,
        pltpu.SemaphoreType.DMA,
        pltpu.SemaphoreType.DMA,
        pltpu.SemaphoreType.DMA,
        pltpu.SemaphoreType.DMA,
    ],
    compiler_params=pltpu.CompilerParams(use_tc_tiling_on_sc=False),
)
def _agg_kernel(src_hbm, dst_hbm, hn_hbm, zrows_hbm, out_hbm,
                src_v, dst_v, rows_v, agg_sp,
                g0, g1, g2, g3, s0, s1, s2, s3):
    c = lax.axis_index("c")
    s = lax.axis_index("s")
    row0 = (c * NS + s) * ROWS_PER_WORKER
    pltpu.sync_copy(src_hbm.at[pl.ds(row0, ROWS_PER_WORKER)], src_v)
    pltpu.sync_copy(dst_hbm.at[pl.ds(row0, ROWS_PER_WORKER)], dst_v)
    pltpu.sync_copy(zrows_hbm, agg_sp.at[pl.ds(s * SLICE, SLICE)])
    plsc.subcore_barrier()
    gsems = (g0, g1, g2, g3)
    ssems = (s0, s1, s2, s3)

    def fire_g(k, b):
        for j in range(RCHUNK):
            pltpu.async_copy(hn_hbm.at[src_v.at[k * RCHUNK + j]],
                             rows_v.at[b].at[pl.ds(j * BATCH, BATCH)], gsems[b])

    def drain_g(b):
        pltpu.make_async_copy(hn_hbm.at[pl.ds(0, RCHUNK * BATCH)],
                              rows_v.at[b], gsems[b]).wait()

    def fire_s(k, b):
        for j in range(RCHUNK):
            pltpu.async_copy(rows_v.at[b].at[pl.ds(j * BATCH, BATCH)],
                             agg_sp.at[dst_v.at[k * RCHUNK + j]],
                             ssems[b], add=True)

    def drain_s(b):
        pltpu.make_async_copy(hn_hbm.at[pl.ds(0, RCHUNK * BATCH)],
                              rows_v.at[b], ssems[b]).wait()

    fire_g(0, 0)
    fire_g(1, 1)
    fire_g(2, 2)
    drain_g(0)
    fire_s(0, 0)
    fire_g(3, 3)
    drain_g(1)
    fire_s(1, 1)

    def step(k, b):
        drain_s((b + 2) % 4)
        fire_g(k + 2, (b + 2) % 4)
        drain_g(b)
        fire_s(k, b)

    def body(p, carry):
        k = 4 * p + 2
        step(k, 2)
        step(k + 1, 3)
        step(k + 2, 0)
        step(k + 3, 1)
        return carry

    lax.fori_loop(0, (CHUNKS - 4) // 4, body, 0)
    drain_s(0)
    drain_g((CHUNKS - 2) % 4)
    fire_s(CHUNKS - 2, (CHUNKS - 2) % 4)
    drain_s(1)
    drain_g((CHUNKS - 1) % 4)
    fire_s(CHUNKS - 1, (CHUNKS - 1) % 4)
    drain_s(2)
    drain_s(3)
    plsc.subcore_barrier()
    pltpu.sync_copy(agg_sp.at[pl.ds(s * SLICE, SLICE)],
                    out_hbm.at[c].at[pl.ds(s * SLICE, SLICE)])


def _prep_body(degp_ref, x_ref, w1_ref, hn_ref, dinv_ref):
    deg = degp_ref[:, 0:1] + degp_ref[:, 1:2] + 1.0
    dinv = 1.0 / jnp.sqrt(deg)
    dinv_ref[...] = dinv
    h = jnp.dot(x_ref[...], w1_ref[...], preferred_element_type=jnp.float32)
    hn_ref[...] = h * dinv


def _mid_body(aggp_ref, hn_ref, dinv_ref, b_ref, w2_ref, out_ref):
    agg = aggp_ref[0, :N, :] + aggp_ref[1, :N, :]
    dinv = dinv_ref[...]
    a = jnp.maximum(dinv * (agg + hn_ref[...]) + b_ref[...], 0.0)
    h = jnp.dot(a, w2_ref[...], preferred_element_type=jnp.float32)
    out_ref[...] = h * dinv


def _fin_body(aggp_ref, hn_ref, dinv_ref, b_ref, wfc_ref, bfc_ref, out_ref):
    agg = aggp_ref[0, :N, :] + aggp_ref[1, :N, :]
    a = jnp.maximum(dinv_ref[...] * (agg + hn_ref[...]) + b_ref[...], 0.0)
    s = a[0:625]
    for i in range(1, 16):
        s = s + a[i * 625:(i + 1) * 625]
    t = s[0:125]
    for i in range(1, 5):
        t = t + s[i * 125:(i + 1) * 125]
    g = jnp.sum(t, axis=0, keepdims=True) * (1.0 / N)
    out_ref[...] = (jnp.sum(g * wfc_ref[...], axis=1, keepdims=True)
                    + bfc_ref[...])


_prep = pl.pallas_call(
    _prep_body,
    out_shape=(jax.ShapeDtypeStruct((N, HID), jnp.float32),
               jax.ShapeDtypeStruct((N, 1), jnp.float32)),
)
_mid = pl.pallas_call(
    _mid_body,
    out_shape=jax.ShapeDtypeStruct((N, HID), jnp.float32),
)
_fin = pl.pallas_call(
    _fin_body,
    out_shape=jax.ShapeDtypeStruct((1, 1), jnp.float32),
)


def kernel(x, edge_index, W1, b1, W2, b2, Wfc, bfc):
    ei32 = edge_index.astype(jnp.int32).reshape(2, EDGE_ROWS, BATCH)
    src2 = ei32[0]
    dst2 = ei32[1]

    zeros1 = jnp.zeros((SLICE,), jnp.float32)
    ones1 = jnp.ones((BATCH,), jnp.float32)
    zrows = jnp.zeros((SLICE, HID), jnp.float32)

    degp = _deg_kernel(dst2, zeros1, ones1)
    degp_col = degp.T[:N]

    hn1, dinv = _prep(degp_col, x, W1)
    agg1 = _agg_kernel(src2, dst2, hn1, zrows)
    hn2 = _mid(agg1, hn1, dinv, b1.reshape(1, HID), W2)
    agg2 = _agg_kernel(src2, dst2, hn2, zrows)
    out = _fin(agg2, hn2, dinv, b2.reshape(1, HID), Wfc.reshape(1, HID),
               bfc.reshape(1, 1))
    return out


# speedup vs baseline: 1.0300x; 1.0300x over previous
import functools

import jax
import jax.numpy as jnp
from jax import lax
from jax.experimental import pallas as pl
from jax.experimental.pallas import tpu as pltpu
from jax.experimental.pallas import tpu_sc as plsc

N = 10000
IN_CH = 128
HID = 64
E = 320000

NC, NS = 2, 16
NW = NC * NS
BATCH = 125
EDGE_ROWS = E // BATCH
ROWS_PER_WORKER = EDGE_ROWS // NW
RCHUNK = 2
CHUNKS = ROWS_PER_WORKER // RCHUNK
NPAD = 10240
SLICE = NPAD // NS

_sc_mesh = plsc.VectorSubcoreMesh(core_axis_name="c", subcore_axis_name="s")


@functools.partial(
    pl.kernel,
    out_type=jax.ShapeDtypeStruct((NC, NPAD), jnp.float32),
    mesh=_sc_mesh,
    scratch_types=[
        pltpu.VMEM((ROWS_PER_WORKER, BATCH), jnp.int32),
        pltpu.VMEM((BATCH,), jnp.float32),
        pltpu.VMEM_SHARED((NPAD,), jnp.float32),
        pltpu.SemaphoreType.DMA,
    ],
    compiler_params=pltpu.CompilerParams(use_tc_tiling_on_sc=False),
)
def _deg_kernel(dst_hbm, zeros_hbm, ones_hbm, out_hbm, idx_v, ones_v, deg_sp,
                sem):
    c = lax.axis_index("c")
    s = lax.axis_index("s")
    row0 = (c * NS + s) * ROWS_PER_WORKER
    pltpu.sync_copy(dst_hbm.at[pl.ds(row0, ROWS_PER_WORKER)], idx_v)
    pltpu.sync_copy(zeros_hbm.at[pl.ds(0, SLICE)],
                    deg_sp.at[pl.ds(s * SLICE, SLICE)])
    pltpu.sync_copy(ones_hbm, ones_v)
    plsc.subcore_barrier()

    def fire(k, carry):
        pltpu.async_copy(ones_v, deg_sp.at[idx_v.at[k]], sem, add=True)
        return carry

    lax.fori_loop(0, ROWS_PER_WORKER, fire, 0)

    def drain(k, carry):
        pltpu.make_async_copy(ones_v, deg_sp.at[idx_v.at[k]], sem).wait()
        return carry

    lax.fori_loop(0, ROWS_PER_WORKER, drain, 0)
    plsc.subcore_barrier()
    pltpu.sync_copy(deg_sp.at[pl.ds(s * SLICE, SLICE)],
                    out_hbm.at[c].at[pl.ds(s * SLICE, SLICE)])


@functools.partial(
    pl.kernel,
    out_type=jax.ShapeDtypeStruct((NC, NPAD, HID), jnp.float32),
    mesh=_sc_mesh,
    scratch_types=[
        pltpu.VMEM((ROWS_PER_WORKER, BATCH), jnp.int32),
        pltpu.VMEM((ROWS_PER_WORKER, BATCH), jnp.int32),
        pltpu.VMEM((3, RCHUNK * BATCH, HID), jnp.float32),
        pltpu.VMEM_SHARED((NPAD, HID), jnp.float32),
        pltpu.SemaphoreType.DMA,
        pltpu.SemaphoreType.DMA,
        pltpu.SemaphoreType.DMA,
        pltpu.SemaphoreType.DMA,
        pltpu.SemaphoreType.DMA,
        pltpu.SemaphoreType.DMA,
    ],
    compiler_params=pltpu.CompilerParams(use_tc_tiling_on_sc=False),
)
def _agg_kernel(src_hbm, dst_hbm, hn_hbm, zrows_hbm, out_hbm,
                src_v, dst_v, rows_v, agg_sp, g0, g1, g2, s0, s1, s2):
    c = lax.axis_index("c")
    s = lax.axis_index("s")
    row0 = (c * NS + s) * ROWS_PER_WORKER
    pltpu.sync_copy(src_hbm.at[pl.ds(row0, ROWS_PER_WORKER)], src_v)
    pltpu.sync_copy(dst_hbm.at[pl.ds(row0, ROWS_PER_WORKER)], dst_v)
    pltpu.sync_copy(zrows_hbm, agg_sp.at[pl.ds(s * SLICE, SLICE)])
    plsc.subcore_barrier()
    gsems = (g0, g1, g2)
    ssems = (s0, s1, s2)

    def fire_g(k, b):
        for j in range(RCHUNK):
            pltpu.async_copy(hn_hbm.at[src_v.at[k * RCHUNK + j]],
                             rows_v.at[b].at[pl.ds(j * BATCH, BATCH)], gsems[b])

    def drain_g(b):
        pltpu.make_async_copy(hn_hbm.at[pl.ds(0, RCHUNK * BATCH)],
                              rows_v.at[b], gsems[b]).wait()

    def fire_s(k, b):
        for j in range(RCHUNK):
            pltpu.async_copy(rows_v.at[b].at[pl.ds(j * BATCH, BATCH)],
                             agg_sp.at[dst_v.at[k * RCHUNK + j]],
                             ssems[b], add=True)

    def drain_s(b):
        pltpu.make_async_copy(hn_hbm.at[pl.ds(0, RCHUNK * BATCH)],
                              rows_v.at[b], ssems[b]).wait()

    fire_g(0, 0)
    fire_g(1, 1)
    fire_g(2, 2)
    drain_g(0)
    fire_s(0, 0)

    def step(k, b):
        drain_s((b + 2) % 3)
        fire_g(k + 2, (b + 2) % 3)
        drain_g(b)
        fire_s(k, b)

    def body(p, carry):
        k = 3 * p + 1
        step(k, 1)
        step(k + 1, 2)
        step(k + 2, 0)
        return carry

    lax.fori_loop(0, (CHUNKS - 4) // 3, body, 0)
    step(CHUNKS - 3, (CHUNKS - 3) % 3)
    drain_g((CHUNKS - 2) % 3)
    fire_s(CHUNKS - 2, (CHUNKS - 2) % 3)
    drain_g((CHUNKS - 1) % 3)
    fire_s(CHUNKS - 1, (CHUNKS - 1) % 3)
    drain_s(0)
    drain_s(1)
    drain_s(2)
    plsc.subcore_barrier()
    pltpu.sync_copy(agg_sp.at[pl.ds(s * SLICE, SLICE)],
                    out_hbm.at[c].at[pl.ds(s * SLICE, SLICE)])


def _prep_body(degp_ref, x_ref, w1_ref, hn_ref, dinv_ref):
    deg = degp_ref[:, 0:1] + degp_ref[:, 1:2] + 1.0
    dinv = 1.0 / jnp.sqrt(deg)
    dinv_ref[...] = dinv
    h = jnp.dot(x_ref[...], w1_ref[...], preferred_element_type=jnp.float32)
    hn_ref[...] = h * dinv


def _mid_body(aggp_ref, hn_ref, dinv_ref, b_ref, w2_ref, out_ref):
    agg = aggp_ref[0, :N, :] + aggp_ref[1, :N, :]
    dinv = dinv_ref[...]
    a = jnp.maximum(dinv * (agg + hn_ref[...]) + b_ref[...], 0.0)
    h = jnp.dot(a, w2_ref[...], preferred_element_type=jnp.float32)
    out_ref[...] = h * dinv


def _fin_body(aggp_ref, hn_ref, dinv_ref, b_ref, wfc_ref, bfc_ref, out_ref):
    agg = aggp_ref[0, :N, :] + aggp_ref[1, :N, :]
    a = jnp.maximum(dinv_ref[...] * (agg + hn_ref[...]) + b_ref[...], 0.0)
    s = a[0:625]
    for i in range(1, 16):
        s = s + a[i * 625:(i + 1) * 625]
    t = s[0:125]
    for i in range(1, 5):
        t = t + s[i * 125:(i + 1) * 125]
    g = jnp.sum(t, axis=0, keepdims=True) * (1.0 / N)
    out_ref[...] = (jnp.sum(g * wfc_ref[...], axis=1, keepdims=True)
                    + bfc_ref[...])


_prep = pl.pallas_call(
    _prep_body,
    out_shape=(jax.ShapeDtypeStruct((N, HID), jnp.float32),
               jax.ShapeDtypeStruct((N, 1), jnp.float32)),
)
_mid = pl.pallas_call(
    _mid_body,
    out_shape=jax.ShapeDtypeStruct((N, HID), jnp.float32),
)
_fin = pl.pallas_call(
    _fin_body,
    out_shape=jax.ShapeDtypeStruct((1, 1), jnp.float32),
)


def kernel(x, edge_index, W1, b1, W2, b2, Wfc, bfc):
    ei32 = edge_index.astype(jnp.int32).reshape(2, EDGE_ROWS, BATCH)
    src2 = ei32[0]
    dst2 = ei32[1]

    zeros1 = jnp.zeros((SLICE,), jnp.float32)
    ones1 = jnp.ones((BATCH,), jnp.float32)
    zrows = jnp.zeros((SLICE, HID), jnp.float32)

    degp = _deg_kernel(dst2, zeros1, ones1)
    degp_col = degp.T[:N]

    hn1, dinv = _prep(degp_col, x, W1)
    agg1 = _agg_kernel(src2, dst2, hn1, zrows)
    hn2 = _mid(agg1, hn1, dinv, b1.reshape(1, HID), W2)
    agg2 = _agg_kernel(src2, dst2, hn2, zrows)
    out = _fin(agg2, hn2, dinv, b2.reshape(1, HID), Wfc.reshape(1, HID),
               bfc.reshape(1, 1))
    return out
